# baseline (speedup 1.0000x reference)
import jax
import jax.numpy as jnp
from jax import lax
from jax.experimental import pallas as pl
from jax.experimental.pallas import tpu as pltpu
from jax.experimental.pallas import tpu_sc as plsc

_B, _S, _V = 2, 2048, 32000
_ROW_OFF = (0 * _S + (_S - 1)) * _V
_NC, _NS = 2, 16
_NW = _NC * _NS
_CHUNK = _V // _NW


def _copy_row(src_hbm, out_hbm, buf):
    wid = lax.axis_index("s") * _NC + lax.axis_index("c")
    base = _ROW_OFF + wid * _CHUNK
    pltpu.sync_copy(src_hbm.at[pl.ds(base, _CHUNK)], buf)
    pltpu.sync_copy(buf, out_hbm.at[pl.ds(wid * _CHUNK, _CHUNK)])


def kernel(logits):
    flat = logits.reshape(-1)
    k = pl.kernel(
        _copy_row,
        out_type=jax.ShapeDtypeStruct((_V,), jnp.float32),
        mesh=plsc.VectorSubcoreMesh(core_axis_name="c", subcore_axis_name="s"),
        scratch_types=[pltpu.VMEM((_CHUNK,), jnp.float32)],
    )
    out = k(flat)
    return out.reshape(1, _V)


# speedup vs baseline: 16.9932x; 16.9932x over previous
import jax
import jax.numpy as jnp
from jax import lax
from jax.experimental import pallas as pl
from jax.experimental.pallas import tpu as pltpu
from jax.experimental.pallas import tpu_sc as plsc

_B, _S, _V = 2, 2048, 32000
_NC, _NS = 2, 16
_NACT = 25
_CCHUNK = _V // _NACT


def _copy_row(src_hbm, out_hbm, buf):
    wid = lax.axis_index("s") * _NC + lax.axis_index("c")

    @pl.when(wid < _NACT)
    def _():
        col = wid * _CCHUNK
        pltpu.sync_copy(src_hbm.at[0, pl.ds(_S - 8, 8), pl.ds(col, _CCHUNK)], buf)
        pltpu.sync_copy(buf.at[7], out_hbm.at[pl.ds(col, _CCHUNK)])


def kernel(logits):
    k = pl.kernel(
        _copy_row,
        out_type=jax.ShapeDtypeStruct((_V,), jnp.float32),
        mesh=plsc.VectorSubcoreMesh(core_axis_name="c", subcore_axis_name="s"),
        scratch_types=[pltpu.VMEM((8, _CCHUNK), jnp.float32)],
    )
    out = k(logits)
    return out.reshape(1, _V)


# speedup vs baseline: 18.7285x; 1.1021x over previous
import jax
import jax.numpy as jnp
from jax import lax
from jax.experimental import pallas as pl
from jax.experimental.pallas import tpu as pltpu
from jax.experimental.pallas import tpu_sc as plsc

_B, _S, _V = 2, 2048, 32000
_NC, _NS = 2, 16
_NACT = 25
_CCHUNK = _V // _NACT


def _copy_row(src_hbm, out_hbm, buf):
    wid = lax.axis_index("s") * _NC + lax.axis_index("c")

    @pl.when(wid < _NACT)
    def _():
        col = wid * _CCHUNK
        pltpu.sync_copy(src_hbm.at[0, pl.ds(_S - 8, 8), pl.ds(col, _CCHUNK)], buf)
        pltpu.sync_copy(buf.at[7], out_hbm.at[0, pl.ds(col, _CCHUNK)])


def kernel(logits):
    k = pl.kernel(
        _copy_row,
        out_type=jax.ShapeDtypeStruct((1, _V), jnp.float32),
        mesh=plsc.VectorSubcoreMesh(core_axis_name="c", subcore_axis_name="s"),
        scratch_types=[pltpu.VMEM((8, _CCHUNK), jnp.float32)],
    )
    return k(logits)


# speedup vs baseline: 18.9704x; 1.0129x over previous
import jax
import jax.numpy as jnp
from jax import lax
from jax.experimental import pallas as pl
from jax.experimental.pallas import tpu as pltpu
from jax.experimental.pallas import tpu_sc as plsc

_B, _S, _V = 2, 2048, 32000
_NC, _NS = 2, 16
_NACT = 25
_CCHUNK = _V // _NACT


def _copy_row(src_hbm, out_hbm, buf):
    wid = lax.axis_index("s")
    for t in (wid, wid + _NS):
        @pl.when(t < _NACT)
        def _():
            col = t * _CCHUNK
            pltpu.sync_copy(src_hbm.at[0, pl.ds(_S - 8, 8), pl.ds(col, _CCHUNK)], buf)
            pltpu.sync_copy(buf.at[7], out_hbm.at[0, pl.ds(col, _CCHUNK)])


def kernel(logits):
    k = pl.kernel(
        _copy_row,
        out_type=jax.ShapeDtypeStruct((1, _V), jnp.float32),
        mesh=plsc.VectorSubcoreMesh(
            core_axis_name="c", subcore_axis_name="s", num_cores=1
        ),
        scratch_types=[pltpu.VMEM((8, _CCHUNK), jnp.float32)],
    )
    return k(logits)


# speedup vs baseline: 20.9089x; 1.1022x over previous
import jax
import jax.numpy as jnp
from jax import lax
from jax.experimental import pallas as pl
from jax.experimental.pallas import tpu as pltpu
from jax.experimental.pallas import tpu_sc as plsc

_B, _S, _V = 2, 2048, 32000
_NC, _NS = 2, 16
_NACT = 25
_CCHUNK = _V // _NACT


def _copy_row(src_hbm, out_hbm, buf):
    pltpu.sync_copy(src_hbm.at[0, pl.ds(_S - 8, 8), :], buf)
    pltpu.sync_copy(buf.at[7], out_hbm.at[0, :])


def kernel(logits):
    k = pl.kernel(
        _copy_row,
        out_type=jax.ShapeDtypeStruct((1, _V), jnp.float32),
        mesh=plsc.ScalarSubcoreMesh(axis_name="c", num_cores=1),
        scratch_types=[pltpu.VMEM_SHARED((8, _V), jnp.float32)],
    )
    return k(logits)
